# Initial kernel scaffold; baseline (speedup 1.0000x reference)
import jax
import jax.numpy as jnp
from jax.experimental import pallas as pl


def kernel(input, target):
    raise NotImplementedError("write your pallas kernel here")


# speedup vs baseline: 15.3575x; 15.3575x over previous
import functools

import jax
import jax.numpy as jnp
from jax import lax
from jax.experimental import pallas as pl
from jax.experimental.pallas import tpu as pltpu
from jax.experimental.pallas import tpu_sc as plsc

NB = 8
NC = 19
NPIX = 512 * 512
NW = 32
PIX_PER_TILE = NPIX // NW
P = 2048
CHUNKS = PIX_PER_TILE // P
LANES = 16
ACC = NC * LANES

_mesh = plsc.VectorSubcoreMesh(core_axis_name="c", subcore_axis_name="s")


@functools.partial(
    pl.kernel,
    mesh=_mesh,
    compiler_params=pltpu.CompilerParams(needs_layout_passes=False),
    out_type=jax.ShapeDtypeStruct((NW * NB * 3 * ACC,), jnp.float32),
    scratch_types=[
        pltpu.VMEM((NC * P,), jnp.float32),
        pltpu.VMEM((P,), jnp.int32),
        pltpu.VMEM((ACC,), jnp.float32),
        pltpu.VMEM((ACC,), jnp.float32),
        pltpu.VMEM((ACC,), jnp.float32),
        pltpu.SemaphoreType.DMA,
    ],
)
def _dice_partials(inp_hbm, tgt_hbm, out_hbm, buf, tbuf, acc2, acc3, stage, sem):
    wid = lax.axis_index("s") * 2 + lax.axis_index("c")
    lane = lax.iota(jnp.int32, LANES)
    zeros = jnp.zeros((LANES,), jnp.float32)
    ones = jnp.ones((LANES,), jnp.float32)

    for b in range(NB):
        for c in range(NC):
            acc2[pl.ds(c * LANES, LANES)] = zeros
            acc3[pl.ds(c * LANES, LANES)] = zeros
        s1 = (zeros,) * NC
        for k in range(CHUNKS):
            off = wid * PIX_PER_TILE + k * P
            for c in range(NC):
                pltpu.sync_copy(
                    inp_hbm.at[pl.ds((b * NC + c) * NPIX + off, P)],
                    buf.at[pl.ds(c * P, P)],
                )
            pltpu.sync_copy(tgt_hbm.at[pl.ds(b * NPIX + off, P)], tbuf)

            def body(i, s1):
                t16 = tbuf[pl.ds(i * LANES, LANES)]
                p16 = lane + i * LANES
                v16 = plsc.load_gather(buf, [t16 * P + p16])
                idx = t16 * LANES + lane
                plsc.addupdate_scatter(acc3, [idx], v16)
                plsc.addupdate_scatter(acc2, [idx], ones)
                return tuple(
                    s1[c] + buf[pl.ds(c * P + i * LANES, LANES)]
                    for c in range(NC)
                )

            s1 = lax.fori_loop(0, P // LANES, body, s1)
        for c in range(NC):
            stage[pl.ds(c * LANES, LANES)] = s1[c]
        base = ((wid * NB + b) * 3) * ACC
        pltpu.sync_copy(stage, out_hbm.at[pl.ds(base, ACC)])
        pltpu.sync_copy(acc2, out_hbm.at[pl.ds(base + ACC, ACC)])
        pltpu.sync_copy(acc3, out_hbm.at[pl.ds(base + 2 * ACC, ACC)])


def kernel(input, target):
    inp3 = input.reshape(NB * NC * NPIX)
    tgt = target.reshape(NB * NPIX).astype(jnp.int32)
    parts = _dice_partials(inp3, tgt).reshape(NW, NB, 3, ACC)
    s = parts.sum(axis=0).reshape(NB, 3, NC, LANES).sum(axis=-1)
    s1, s2, s3 = s[:, 0], s[:, 1], s[:, 2]
    r = (s3 + 1.0) / (s1 + s2 + 1.0)
    return jnp.float32(NC) - 0.25 * r.sum()


# speedup vs baseline: 38.4659x; 2.5047x over previous
import functools

import jax
import jax.numpy as jnp
from jax import lax
from jax.experimental import pallas as pl
from jax.experimental.pallas import tpu as pltpu
from jax.experimental.pallas import tpu_sc as plsc

NB = 8
NC = 19
NPIX = 512 * 512
NW = 32
PIX_PER_TILE = NPIX // NW
P = 2048
CHUNKS = PIX_PER_TILE // P
LANES = 16
ACC = NC * LANES

_mesh = plsc.VectorSubcoreMesh(core_axis_name="c", subcore_axis_name="s")


@functools.partial(
    pl.kernel,
    mesh=_mesh,
    compiler_params=pltpu.CompilerParams(needs_layout_passes=False),
    out_type=jax.ShapeDtypeStruct((NW * NB * 3 * ACC,), jnp.float32),
    scratch_types=[
        pltpu.VMEM((NC * P,), jnp.float32),
        pltpu.VMEM((NC * P,), jnp.float32),
        pltpu.VMEM((P,), jnp.int32),
        pltpu.VMEM((P,), jnp.int32),
        pltpu.VMEM((ACC,), jnp.float32),
        pltpu.VMEM((ACC,), jnp.float32),
        pltpu.VMEM((ACC,), jnp.float32),
        pltpu.SemaphoreType.DMA,
        pltpu.SemaphoreType.DMA,
    ],
)
def _dice_partials(
    inp_hbm, tgt_hbm, out_hbm, buf0, buf1, tbuf0, tbuf1, acc2, acc3, stage,
    sem0, sem1,
):
    wid = lax.axis_index("s") * 2 + lax.axis_index("c")
    lane = lax.iota(jnp.int32, LANES)
    zeros = jnp.zeros((LANES,), jnp.float32)
    ones = jnp.ones((LANES,), jnp.float32)
    slots = ((buf0, tbuf0, sem0), (buf1, tbuf1, sem1))

    def fire(b, k, slot):
        buf, tbuf, sem = slot
        off = wid * PIX_PER_TILE + k * P
        cps = [
            pltpu.make_async_copy(
                inp_hbm.at[pl.ds((b * NC + c) * NPIX + off, P)],
                buf.at[pl.ds(c * P, P)],
                sem,
            )
            for c in range(NC)
        ]
        cps.append(
            pltpu.make_async_copy(
                tgt_hbm.at[pl.ds(b * NPIX + off, P)], tbuf, sem
            )
        )
        for cp in cps:
            cp.start()
        return cps

    for b in range(NB):
        for c in range(NC):
            acc2[pl.ds(c * LANES, LANES)] = zeros
            acc3[pl.ds(c * LANES, LANES)] = zeros
        s1 = (zeros,) * NC
        pend = fire(b, 0, slots[0])
        for k in range(CHUNKS):
            buf, tbuf, _ = slots[k % 2]
            cur = pend
            if k + 1 < CHUNKS:
                pend = fire(b, k + 1, slots[(k + 1) % 2])
            for cp in cur:
                cp.wait()

            def body(i, s1):
                t16 = tbuf[pl.ds(i * LANES, LANES)]
                p16 = lane + i * LANES
                v16 = plsc.load_gather(buf, [t16 * P + p16])
                idx = t16 * LANES + lane
                plsc.addupdate_scatter(acc3, [idx], v16)
                plsc.addupdate_scatter(acc2, [idx], ones)
                return tuple(
                    s1[c] + buf[pl.ds(c * P + i * LANES, LANES)]
                    for c in range(NC)
                )

            s1 = lax.fori_loop(0, P // LANES, body, s1)
        for c in range(NC):
            stage[pl.ds(c * LANES, LANES)] = s1[c]
        base = ((wid * NB + b) * 3) * ACC
        pltpu.sync_copy(stage, out_hbm.at[pl.ds(base, ACC)])
        pltpu.sync_copy(acc2, out_hbm.at[pl.ds(base + ACC, ACC)])
        pltpu.sync_copy(acc3, out_hbm.at[pl.ds(base + 2 * ACC, ACC)])


def kernel(input, target):
    inp3 = input.reshape(NB * NC * NPIX)
    tgt = target.reshape(NB * NPIX).astype(jnp.int32)
    parts = _dice_partials(inp3, tgt).reshape(NW, NB, 3, ACC)
    s = parts.sum(axis=0).reshape(NB, 3, NC, LANES).sum(axis=-1)
    s1, s2, s3 = s[:, 0], s[:, 1], s[:, 2]
    r = (s3 + 1.0) / (s1 + s2 + 1.0)
    return jnp.float32(NC) - 0.25 * r.sum()


# speedup vs baseline: 44.6681x; 1.1612x over previous
import functools

import jax
import jax.numpy as jnp
from jax import lax
from jax.experimental import pallas as pl
from jax.experimental.pallas import tpu as pltpu
from jax.experimental.pallas import tpu_sc as plsc

NB = 8
NC = 19
NPIX = 512 * 512
NW = 32
PIX_PER_TILE = NPIX // NW
P = 2048
CHUNKS = PIX_PER_TILE // P
TOT = NB * CHUNKS
CP = NC * P
LANES = 16
ACC = NC * LANES

_mesh = plsc.VectorSubcoreMesh(core_axis_name="c", subcore_axis_name="s")


@functools.partial(
    pl.kernel,
    mesh=_mesh,
    compiler_params=pltpu.CompilerParams(needs_layout_passes=False),
    out_type=jax.ShapeDtypeStruct((NW * NB * 3 * ACC,), jnp.float32),
    scratch_types=[
        pltpu.VMEM((2 * CP,), jnp.float32),
        pltpu.VMEM((2 * P,), jnp.int32),
        pltpu.VMEM((ACC,), jnp.float32),
        pltpu.VMEM((ACC,), jnp.float32),
        pltpu.VMEM((ACC,), jnp.float32),
        pltpu.SemaphoreType.DMA,
        pltpu.SemaphoreType.DMA,
    ],
)
def _dice_partials(
    inp_hbm, tgt_hbm, out_hbm, buf, tbuf, acc2, acc3, stage, sem0, sem1
):
    wid = lax.axis_index("s") * 2 + lax.axis_index("c")
    lane = lax.iota(jnp.int32, LANES)
    zeros = jnp.zeros((LANES,), jnp.float32)
    ones = jnp.ones((LANES,), jnp.float32)

    def fire(t, sem):
        b = t // CHUNKS
        off = wid * PIX_PER_TILE + (t % CHUNKS) * P
        par = t % 2
        for c in range(NC):
            pltpu.make_async_copy(
                inp_hbm.at[pl.ds((b * NC + c) * NPIX + off, P)],
                buf.at[pl.ds(par * CP + c * P, P)],
                sem,
            ).start()
        pltpu.make_async_copy(
            tgt_hbm.at[pl.ds(b * NPIX + off, P)],
            tbuf.at[pl.ds(par * P, P)],
            sem,
        ).start()

    def drain(t, sem):
        par = t % 2
        pltpu.make_async_copy(
            inp_hbm.at[pl.ds(0, CP)], buf.at[pl.ds(par * CP, CP)], sem
        ).wait()
        pltpu.make_async_copy(
            tgt_hbm.at[pl.ds(0, P)], tbuf.at[pl.ds(par * P, P)], sem
        ).wait()

    fire(0, sem0)

    def outer(t, s1):
        par = t % 2
        reset = (t % CHUNKS) == 0
        s1 = tuple(jnp.where(reset, zeros, s) for s in s1)

        @pl.when(reset)
        def _():
            for c in range(NC):
                acc2[pl.ds(c * LANES, LANES)] = zeros
                acc3[pl.ds(c * LANES, LANES)] = zeros

        @pl.when(jnp.logical_and(t + 1 < TOT, par == 0))
        def _():
            fire(t + 1, sem1)

        @pl.when(jnp.logical_and(t + 1 < TOT, par == 1))
        def _():
            fire(t + 1, sem0)

        @pl.when(par == 0)
        def _():
            drain(t, sem0)

        @pl.when(par == 1)
        def _():
            drain(t, sem1)

        vbase = par * CP
        tbase = par * P

        def body(i, s1):
            t16 = tbuf[pl.ds(tbase + i * LANES, LANES)]
            p16 = lane + i * LANES
            v16 = plsc.load_gather(buf, [vbase + t16 * P + p16])
            idx = t16 * LANES + lane
            plsc.addupdate_scatter(acc3, [idx], v16)
            plsc.addupdate_scatter(acc2, [idx], ones)
            return tuple(
                s1[c] + buf[pl.ds(vbase + c * P + i * LANES, LANES)]
                for c in range(NC)
            )

        s1 = lax.fori_loop(0, P // LANES, body, s1, unroll=2)

        @pl.when((t % CHUNKS) == CHUNKS - 1)
        def _():
            for c in range(NC):
                stage[pl.ds(c * LANES, LANES)] = s1[c]
            b = t // CHUNKS
            base = ((wid * NB + b) * 3) * ACC
            pltpu.sync_copy(stage, out_hbm.at[pl.ds(base, ACC)])
            pltpu.sync_copy(acc2, out_hbm.at[pl.ds(base + ACC, ACC)])
            pltpu.sync_copy(acc3, out_hbm.at[pl.ds(base + 2 * ACC, ACC)])

        return s1

    lax.fori_loop(0, TOT, outer, (zeros,) * NC)


def kernel(input, target):
    inp3 = input.reshape(NB * NC * NPIX)
    tgt = target.reshape(NB * NPIX).astype(jnp.int32)
    parts = _dice_partials(inp3, tgt).reshape(NW, NB, 3, ACC)
    s = parts.sum(axis=0).reshape(NB, 3, NC, LANES).sum(axis=-1)
    s1, s2, s3 = s[:, 0], s[:, 1], s[:, 2]
    r = (s3 + 1.0) / (s1 + s2 + 1.0)
    return jnp.float32(NC) - 0.25 * r.sum()


# speedup vs baseline: 98.7724x; 2.2113x over previous
import functools

import jax
import jax.numpy as jnp
from jax import lax
from jax.experimental import pallas as pl
from jax.experimental.pallas import tpu as pltpu
from jax.experimental.pallas import tpu_sc as plsc

NB = 8
NC = 19
H = 512
W = 512
NW = 32
ROWS_PER_TILE = H // NW
R = 8
CW = 256
CHUNKS = 4
TOT = NB * CHUNKS
BROWS = NC * R
LANES = 16
ACC = NC * LANES

_mesh = plsc.VectorSubcoreMesh(core_axis_name="c", subcore_axis_name="s")


@functools.partial(
    pl.kernel,
    mesh=_mesh,
    compiler_params=pltpu.CompilerParams(needs_layout_passes=False),
    out_type=jax.ShapeDtypeStruct((NW * NB * 3 * ACC,), jnp.float32),
    scratch_types=[
        pltpu.VMEM((2 * BROWS, CW), jnp.float32),
        pltpu.VMEM((2 * R, CW), jnp.int32),
        pltpu.VMEM((ACC,), jnp.float32),
        pltpu.VMEM((ACC,), jnp.float32),
        pltpu.VMEM((ACC,), jnp.float32),
        pltpu.SemaphoreType.DMA,
        pltpu.SemaphoreType.DMA,
    ],
)
def _dice_partials(
    inp_hbm, tgt_hbm, out_hbm, buf, tbuf, acc2, acc3, stage, sem0, sem1
):
    wid = lax.axis_index("s") * 2 + lax.axis_index("c")
    lane = lax.iota(jnp.int32, LANES)
    zeros = jnp.zeros((LANES,), jnp.float32)
    ones = jnp.ones((LANES,), jnp.float32)

    def chunk_coords(t):
        b = t // CHUNKS
        k = t % CHUNKS
        h0 = wid * ROWS_PER_TILE + (k // 2) * R
        w0 = (k % 2) * CW
        return b, h0, w0

    def fire(t, sem):
        b, h0, w0 = chunk_coords(t)
        par = t % 2
        for c in range(NC):
            pltpu.make_async_copy(
                inp_hbm.at[b, c, pl.ds(h0, R), pl.ds(w0, CW)],
                buf.at[pl.ds(par * BROWS + c * R, R), :],
                sem,
            ).start()
        pltpu.make_async_copy(
            tgt_hbm.at[b, 0, pl.ds(h0, R), pl.ds(w0, CW)],
            tbuf.at[pl.ds(par * R, R), :],
            sem,
        ).start()

    def drain(t, sem):
        par = t % 2
        pltpu.make_async_copy(
            inp_hbm.at[0, 0, pl.ds(0, BROWS), pl.ds(0, CW)],
            buf.at[pl.ds(par * BROWS, BROWS), :],
            sem,
        ).wait()
        pltpu.make_async_copy(
            tgt_hbm.at[0, 0, pl.ds(0, R), pl.ds(0, CW)],
            tbuf.at[pl.ds(par * R, R), :],
            sem,
        ).wait()

    fire(0, sem0)

    def outer(t, s1):
        par = t % 2
        reset = (t % CHUNKS) == 0
        s1 = tuple(jnp.where(reset, zeros, s) for s in s1)

        @pl.when(reset)
        def _():
            for c in range(NC):
                acc2[pl.ds(c * LANES, LANES)] = zeros
                acc3[pl.ds(c * LANES, LANES)] = zeros

        @pl.when(jnp.logical_and(t + 1 < TOT, par == 0))
        def _():
            fire(t + 1, sem1)

        @pl.when(jnp.logical_and(t + 1 < TOT, par == 1))
        def _():
            fire(t + 1, sem0)

        @pl.when(par == 0)
        def _():
            drain(t, sem0)

        @pl.when(par == 1)
        def _():
            drain(t, sem1)

        vbase = par * BROWS
        tbase = par * R

        def body(i, s1):
            r = i >> 4
            j = i & 15
            col = j * LANES
            t16 = tbuf[tbase + r, pl.ds(col, LANES)]
            w16 = lane + col
            v16 = plsc.load_gather(buf, [vbase + t16 * R + r, w16])
            idx = t16 * LANES + lane
            plsc.addupdate_scatter(acc3, [idx], v16)
            plsc.addupdate_scatter(acc2, [idx], ones)
            return tuple(
                s1[c] + buf[vbase + c * R + r, pl.ds(col, LANES)]
                for c in range(NC)
            )

        s1 = lax.fori_loop(0, R * CW // LANES, body, s1, unroll=2)

        @pl.when((t % CHUNKS) == CHUNKS - 1)
        def _():
            for c in range(NC):
                stage[pl.ds(c * LANES, LANES)] = s1[c]
            b = t // CHUNKS
            base = ((wid * NB + b) * 3) * ACC
            pltpu.sync_copy(stage, out_hbm.at[pl.ds(base, ACC)])
            pltpu.sync_copy(acc2, out_hbm.at[pl.ds(base + ACC, ACC)])
            pltpu.sync_copy(acc3, out_hbm.at[pl.ds(base + 2 * ACC, ACC)])

        return s1

    lax.fori_loop(0, TOT, outer, (zeros,) * NC)


def kernel(input, target):
    tgt = target.astype(jnp.int32)
    parts = _dice_partials(input, tgt).reshape(NW, NB, 3, ACC)
    s = parts.sum(axis=0).reshape(NB, 3, NC, LANES).sum(axis=-1)
    s1, s2, s3 = s[:, 0], s[:, 1], s[:, 2]
    r = (s3 + 1.0) / (s1 + s2 + 1.0)
    return jnp.float32(NC) - 0.25 * r.sum()
